# Initial kernel scaffold; baseline (speedup 1.0000x reference)
import jax
import jax.numpy as jnp
from jax.experimental import pallas as pl


def kernel(X, edges, W1, b1, W2, b2):
    raise NotImplementedError("write your pallas kernel here")


# speedup vs baseline: 15.9050x; 15.9050x over previous
import functools

import jax
import jax.numpy as jnp
from jax import lax
from jax.experimental import pallas as pl
from jax.experimental.pallas import tpu as pltpu
from jax.experimental.pallas import tpu_sc as plsc

N = 10000
D = 128
E = 320000

NC = 2
NS = 16
NW = NC * NS
CHUNK = 128
NCHUNK = E // CHUNK
BASE = NCHUNK // NW
EXTRA = NCHUNK - BASE * NW

ROWS_T = N // NS
HSLICE = 624

_mesh = plsc.VectorSubcoreMesh(core_axis_name="c", subcore_axis_name="s")


def _zero_vmem_2d(ref, nrows):
    z = jnp.zeros((16,), jnp.float32)
    def body(r, _):
        for c in range(D // 16):
            ref[r, pl.ds(c * 16, 16)] = z
        return 0
    lax.fori_loop(0, nrows, body, 0)


@functools.partial(
    pl.kernel,
    out_type=jax.ShapeDtypeStruct((NC * N,), jnp.float32),
    mesh=_mesh,
    scratch_types=[
        pltpu.VMEM((CHUNK,), jnp.int32),
        pltpu.VMEM((CHUNK,), jnp.float32),
        pltpu.VMEM((HSLICE + 16, ), jnp.float32),
        pltpu.VMEM_SHARED((N,), jnp.float32),
        pltpu.SemaphoreType.DMA,
    ],
)
def _sc_degree(dst_hbm, out_hbm, dst_v, ones_v, zbuf, acc, sem):
    cid = lax.axis_index("c")
    sid = lax.axis_index("s")
    wid = sid * NC + cid

    one = jnp.ones((16,), jnp.float32)
    zero = jnp.zeros((16,), jnp.float32)
    for i in range(CHUNK // 16):
        ones_v[pl.ds(i * 16, 16)] = one
    for i in range((HSLICE + 16) // 16):
        zbuf[pl.ds(i * 16, 16)] = zero

    o = sid * HSLICE
    pltpu.sync_copy(zbuf.at[pl.ds(0, HSLICE)], acc.at[pl.ds(o, HSLICE)])
    @pl.when(sid == NS - 1)
    def _():
        pltpu.sync_copy(zbuf.at[pl.ds(0, 16)], acc.at[pl.ds(NS * HSLICE, 16)])
    plsc.subcore_barrier()

    def body(j, _):
        off = (wid * BASE + j) * CHUNK
        pltpu.sync_copy(dst_hbm.at[pl.ds(off, CHUNK)], dst_v)
        pltpu.sync_copy(ones_v, acc.at[dst_v], add=True)
        return 0
    lax.fori_loop(0, BASE, body, 0)

    @pl.when(wid < EXTRA)
    def _():
        off = (NW * BASE + wid) * CHUNK
        pltpu.sync_copy(dst_hbm.at[pl.ds(off, CHUNK)], dst_v)
        pltpu.sync_copy(ones_v, acc.at[dst_v], add=True)

    plsc.subcore_barrier()
    pltpu.sync_copy(acc.at[pl.ds(o, HSLICE)], zbuf.at[pl.ds(0, HSLICE)])
    pltpu.sync_copy(zbuf.at[pl.ds(0, HSLICE)],
                    out_hbm.at[pl.ds(cid * N + o, HSLICE)])
    @pl.when(sid == NS - 1)
    def _():
        pltpu.sync_copy(acc.at[pl.ds(NS * HSLICE, 16)],
                        zbuf.at[pl.ds(HSLICE, 16)])
        pltpu.sync_copy(zbuf.at[pl.ds(HSLICE, 16)],
                        out_hbm.at[pl.ds(cid * N + NS * HSLICE, 16)])


@functools.partial(
    pl.kernel,
    out_type=jax.ShapeDtypeStruct((NC, N, D), jnp.float32),
    mesh=_mesh,
    scratch_types=[
        pltpu.VMEM((CHUNK,), jnp.int32),
        pltpu.VMEM((CHUNK,), jnp.int32),
        pltpu.VMEM((CHUNK, D), jnp.float32),
        pltpu.VMEM_SHARED((N, D), jnp.float32),
        pltpu.SemaphoreType.DMA,
    ],
)
def _sc_propagate(h_hbm, src_hbm, dst_hbm, out_hbm, src_v, dst_v, rows_v,
                  acc, sem):
    cid = lax.axis_index("c")
    sid = lax.axis_index("s")
    wid = sid * NC + cid

    _zero_vmem_2d(rows_v, CHUNK)
    r0 = sid * HSLICE
    for k in range(HSLICE // CHUNK):
        pltpu.sync_copy(rows_v, acc.at[pl.ds(r0 + k * CHUNK, CHUNK)])
    rem = HSLICE - (HSLICE // CHUNK) * CHUNK
    pltpu.sync_copy(rows_v.at[pl.ds(0, rem)],
                    acc.at[pl.ds(r0 + HSLICE - rem, rem)])
    @pl.when(sid == NS - 1)
    def _():
        pltpu.sync_copy(rows_v.at[pl.ds(0, 16)],
                        acc.at[pl.ds(NS * HSLICE, 16)])
    plsc.subcore_barrier()

    def body(j, _):
        off = (wid * BASE + j) * CHUNK
        pltpu.sync_copy(src_hbm.at[pl.ds(off, CHUNK)], src_v)
        pltpu.sync_copy(dst_hbm.at[pl.ds(off, CHUNK)], dst_v)
        pltpu.async_copy(h_hbm.at[src_v], rows_v, sem).wait()
        pltpu.sync_copy(rows_v, acc.at[dst_v], add=True)
        return 0
    lax.fori_loop(0, BASE, body, 0)

    @pl.when(wid < EXTRA)
    def _():
        off = (NW * BASE + wid) * CHUNK
        pltpu.sync_copy(src_hbm.at[pl.ds(off, CHUNK)], src_v)
        pltpu.sync_copy(dst_hbm.at[pl.ds(off, CHUNK)], dst_v)
        pltpu.async_copy(h_hbm.at[src_v], rows_v, sem).wait()
        pltpu.sync_copy(rows_v, acc.at[dst_v], add=True)

    plsc.subcore_barrier()
    for k in range(HSLICE // CHUNK):
        pltpu.sync_copy(acc.at[pl.ds(r0 + k * CHUNK, CHUNK)], rows_v)
        pltpu.sync_copy(rows_v, out_hbm.at[cid, pl.ds(r0 + k * CHUNK, CHUNK)])
    pltpu.sync_copy(acc.at[pl.ds(r0 + HSLICE - rem, rem)],
                    rows_v.at[pl.ds(0, rem)])
    pltpu.sync_copy(rows_v.at[pl.ds(0, rem)],
                    out_hbm.at[cid, pl.ds(r0 + HSLICE - rem, rem)])
    @pl.when(sid == NS - 1)
    def _():
        pltpu.sync_copy(acc.at[pl.ds(NS * HSLICE, 16)],
                        rows_v.at[pl.ds(16, 16)])
        pltpu.sync_copy(rows_v.at[pl.ds(16, 16)],
                        out_hbm.at[cid, pl.ds(NS * HSLICE, 16)])


RB = 1000
GRID = N // RB


def _tc1_body(x_ref, w_ref, d0_ref, d1_ref, hp_ref, dinv_ref):
    deg = d0_ref[...] + d1_ref[...] + 1.0
    dinv = lax.rsqrt(deg)
    dinv_ref[...] = dinv
    h = jnp.dot(x_ref[...], w_ref[...], preferred_element_type=jnp.float32)
    hp_ref[...] = h * dinv


def _tc2_body(a0_ref, a1_ref, hp_ref, dinv_ref, b_ref, w_ref, out_ref):
    dinv = dinv_ref[...]
    z = dinv * (a0_ref[...] + a1_ref[...] + hp_ref[...]) + b_ref[...]
    z = jnp.maximum(z, 0.0)
    h = jnp.dot(z, w_ref[...], preferred_element_type=jnp.float32)
    out_ref[...] = h * dinv


def _tc3_body(a0_ref, a1_ref, hp_ref, dinv_ref, b_ref, out_ref):
    out_ref[...] = (dinv_ref[...] * (a0_ref[...] + a1_ref[...] + hp_ref[...])
                    + b_ref[...])


_row_spec = pl.BlockSpec((RB, D), lambda i: (i, 0))
_col_spec = pl.BlockSpec((RB, 1), lambda i: (i, 0))
_w_spec = pl.BlockSpec((D, D), lambda i: (0, 0))
_b_spec = pl.BlockSpec((1, D), lambda i: (0, 0))

_tc1 = pl.pallas_call(
    _tc1_body,
    grid=(GRID,),
    in_specs=[_row_spec, _w_spec, _col_spec, _col_spec],
    out_specs=[_row_spec, _col_spec],
    out_shape=[jax.ShapeDtypeStruct((N, D), jnp.float32),
               jax.ShapeDtypeStruct((N, 1), jnp.float32)],
)

_tc2 = pl.pallas_call(
    _tc2_body,
    grid=(GRID,),
    in_specs=[_row_spec, _row_spec, _row_spec, _col_spec, _b_spec, _w_spec],
    out_specs=_row_spec,
    out_shape=jax.ShapeDtypeStruct((N, D), jnp.float32),
)

_tc3 = pl.pallas_call(
    _tc3_body,
    grid=(GRID,),
    in_specs=[_row_spec, _row_spec, _row_spec, _col_spec, _b_spec],
    out_specs=_row_spec,
    out_shape=jax.ShapeDtypeStruct((N, D), jnp.float32),
)


def kernel(X, edges, W1, b1, W2, b2):
    src = edges[0].astype(jnp.int32)
    dst = edges[1].astype(jnp.int32)

    degp = _sc_degree(dst)
    deg0 = degp[:N].reshape(N, 1)
    deg1 = degp[N:].reshape(N, 1)

    h1p, dinv = _tc1(X, W1, deg0, deg1)

    a1 = _sc_propagate(h1p, src, dst)
    h2p = _tc2(a1[0], a1[1], h1p, dinv, b1.reshape(1, D), W2)

    a2 = _sc_propagate(h2p, src, dst)
    out = _tc3(a2[0], a2[1], h2p, dinv, b2.reshape(1, D))
    return out
